# Initial kernel scaffold; baseline (speedup 1.0000x reference)
import jax
import jax.numpy as jnp
from jax.experimental import pallas as pl


def kernel(x, edge_index, W1, b1, W2, b2, Wl, bl):
    raise NotImplementedError("write your pallas kernel here")


# speedup vs baseline: 34.0687x; 34.0687x over previous
import functools

import jax
import jax.numpy as jnp
from jax import lax
from jax.experimental import pallas as pl
from jax.experimental.pallas import tpu as pltpu
from jax.experimental.pallas import tpu_sc as plsc

N = 10000
NPAD = 10240
E = 320000
D = 128

NC = 2
NS = 16
NW = NC * NS
EPW = E // NW
CH = 80
NCH = EPW // CH
RPS = NPAD // NS

_mesh = plsc.VectorSubcoreMesh(core_axis_name="c", subcore_axis_name="s")


def _fill(buf, nrows, ncols16, val):
  v = jnp.full((16,), val, jnp.float32)

  @pl.loop(0, nrows)
  def _(i):
    for j in range(ncols16):
      buf[i, pl.ds(j * 16, 16)] = v


@functools.partial(
    pl.kernel,
    out_type=jax.ShapeDtypeStruct((NW, NPAD), jnp.float32),
    mesh=_mesh,
    scratch_types=[
        pltpu.VMEM((NCH, CH), jnp.int32),
        pltpu.VMEM((NPAD,), jnp.float32),
    ],
    compiler_params=pltpu.CompilerParams(needs_layout_passes=False),
)
def _deg_kernel(ei_hbm, out_hbm, idx_v, hist):
  c = lax.axis_index("c")
  s = lax.axis_index("s")
  wid = s * NC + c

  zeros16 = jnp.zeros((16,), jnp.float32)

  @pl.loop(0, NPAD // 16, unroll=8)
  def _(i):
    hist[pl.ds(i * 16, 16)] = zeros16

  pltpu.sync_copy(ei_hbm.at[1, wid], idx_v)
  ones16 = jnp.full((16,), 1.0, jnp.float32)

  @pl.loop(0, NCH, unroll=4)
  def _(j):
    for m in range(CH // 16):
      idx16 = idx_v[j, pl.ds(m * 16, 16)]
      plsc.addupdate_scatter(hist, [idx16], ones16)

  pltpu.sync_copy(hist, out_hbm.at[wid])


@functools.partial(
    pl.kernel,
    out_type=jax.ShapeDtypeStruct((NC, NPAD, D), jnp.float32),
    mesh=_mesh,
    scratch_types=[
        pltpu.VMEM((EPW,), jnp.int32),
        pltpu.VMEM((NCH, CH), jnp.int32),
        pltpu.VMEM((CH, D), jnp.float32),
        pltpu.VMEM((CH, D), jnp.float32),
        pltpu.VMEM_SHARED((NPAD, D), jnp.float32),
        pltpu.SemaphoreType.DMA,
        pltpu.SemaphoreType.DMA,
        pltpu.SemaphoreType.DMA,
    ],
)
def _agg_kernel(h_hbm, src_hbm, ei_hbm, out_hbm,
                src_v, dst_v, buf0, buf1, acc, sem0, sem1, zsem):
  c = lax.axis_index("c")
  s = lax.axis_index("s")
  wid = s * NC + c
  _fill(buf0, CH, D // 16, 0.0)
  for k in range(RPS // CH):
    pltpu.async_copy(buf0, acc.at[pl.ds(s * RPS + k * CH, CH)], zsem)
  pltpu.async_copy(src_hbm.at[wid], src_v, sem0)
  pltpu.async_copy(ei_hbm.at[1, wid], dst_v, sem1)
  pltpu.make_async_copy(src_hbm.at[wid], src_v, sem0).wait()
  pltpu.make_async_copy(ei_hbm.at[1, wid], dst_v, sem1).wait()
  for k in range(RPS // CH):
    pltpu.make_async_copy(buf0, acc.at[pl.ds(s * RPS + k * CH, CH)], zsem).wait()
  pltpu.async_copy(h_hbm.at[src_v.at[pl.ds(0, CH)]], buf0, sem0)
  pltpu.async_copy(h_hbm.at[src_v.at[pl.ds(CH, CH)]], buf1, sem1)
  plsc.subcore_barrier()

  @pl.loop(0, NCH - 1, step=2)
  def _(j):
    pltpu.make_async_copy(h_hbm.at[src_v.at[pl.ds(j * CH, CH)]], buf0, sem0).wait()
    pltpu.sync_copy(buf0, acc.at[dst_v.at[j]], add=True)
    pltpu.async_copy(h_hbm.at[src_v.at[pl.ds((j + 2) * CH, CH)]], buf0, sem0)
    pltpu.make_async_copy(h_hbm.at[src_v.at[pl.ds((j + 1) * CH, CH)]], buf1, sem1).wait()
    pltpu.sync_copy(buf1, acc.at[dst_v.at[j + 1]], add=True)

    @pl.when(j + 3 < NCH)
    def _():
      pltpu.async_copy(h_hbm.at[src_v.at[pl.ds((j + 3) * CH, CH)]], buf1, sem1)

  pltpu.make_async_copy(h_hbm.at[src_v.at[pl.ds((NCH - 1) * CH, CH)]], buf0, sem0).wait()
  pltpu.sync_copy(buf0, acc.at[dst_v.at[NCH - 1]], add=True)

  plsc.subcore_barrier()
  pltpu.sync_copy(acc.at[pl.ds(s * RPS, RPS)],
                  out_hbm.at[c, pl.ds(s * RPS, RPS)])


BN = 2048
_GRID = (N + BN - 1) // BN


def _dis_of(dp_ref):
  deg = jnp.sum(dp_ref[...], axis=0) + 1.0
  return lax.rsqrt(deg).reshape(BN, 1)


def _tc1_body(x_ref, w_ref, dp_ref, o_ref):
  dis = _dis_of(dp_ref)
  o_ref[...] = dis * jnp.dot(x_ref[...], w_ref[...],
                             preferred_element_type=jnp.float32)


def _tc2_body(ap_ref, hp_ref, dp_ref, b_ref, w_ref, o_ref):
  dis = _dis_of(dp_ref)
  ssum = ap_ref[0] + ap_ref[1] + hp_ref[...]
  h1 = jnp.maximum(dis * ssum + b_ref[...], 0.0)
  o_ref[...] = dis * jnp.dot(h1, w_ref[...],
                             preferred_element_type=jnp.float32)


def _tc3_body(ap_ref, hp_ref, dp_ref, b_ref, w_ref, bl_ref, o_ref):
  dis = _dis_of(dp_ref)
  h2 = dis * (ap_ref[0] + ap_ref[1] + hp_ref[...]) + b_ref[...]
  o_ref[...] = jnp.dot(h2, w_ref[...],
                       preferred_element_type=jnp.float32) + bl_ref[...]


_row_spec = pl.BlockSpec((BN, D), lambda i: (i, 0))
_ap_spec = pl.BlockSpec((NC, BN, D), lambda i: (0, i, 0))
_dp_spec = pl.BlockSpec((NW, BN), lambda i: (0, i))
_w_spec = pl.BlockSpec((D, D), lambda i: (0, 0))
_b_spec = pl.BlockSpec((1, D), lambda i: (0, 0))
_out_f32 = jax.ShapeDtypeStruct((N, D), jnp.float32)

_tc1 = pl.pallas_call(
    _tc1_body, grid=(_GRID,),
    in_specs=[_row_spec, _w_spec, _dp_spec],
    out_specs=_row_spec, out_shape=_out_f32)

_tc2 = pl.pallas_call(
    _tc2_body, grid=(_GRID,),
    in_specs=[_ap_spec, _row_spec, _dp_spec, _b_spec, _w_spec],
    out_specs=_row_spec, out_shape=_out_f32)

_tc3 = pl.pallas_call(
    _tc3_body, grid=(_GRID,),
    in_specs=[_ap_spec, _row_spec, _dp_spec, _b_spec, _w_spec, _b_spec],
    out_specs=_row_spec, out_shape=_out_f32)


@jax.jit
def kernel(x, edge_index, W1, b1, W2, b2, Wl, bl):
  ei32 = edge_index.astype(jnp.int32)
  ei4 = ei32.reshape(2, NW, NCH, CH)
  src = ei32[0].reshape(NW, EPW)
  b1r = b1.reshape(1, D)
  b2r = b2.reshape(1, D)
  blr = bl.reshape(1, D)

  dp = _deg_kernel(ei4)
  h1p = _tc1(x, W1, dp)
  ap1 = _agg_kernel(h1p, src, ei4)
  h2p = _tc2(ap1, h1p, dp, b1r, W2)
  ap2 = _agg_kernel(h2p, src, ei4)
  out = _tc3(ap2, h2p, dp, b2r, Wl, blr)
  return (out,)
